# baseline (speedup 1.0000x reference)
import functools

import jax
import jax.numpy as jnp
from jax import lax
from jax.experimental import pallas as pl
from jax.experimental.pallas import tpu as pltpu
from jax.experimental.pallas import tpu_sc as plsc

NC = 2
NS = 16
NW = NC * NS
GW = 16
D = 32
SL = 8


def _make_sc_kernel(B, b_per_w, n_groups):
    mesh = plsc.VectorSubcoreMesh(core_axis_name="c", subcore_axis_name="s")

    @functools.partial(
        pl.kernel,
        mesh=mesh,
        compiler_params=pltpu.CompilerParams(
            use_tc_tiling_on_sc=True, needs_layout_passes=False),
        out_type=jax.ShapeDtypeStruct((B,), jnp.float32),
        scratch_types=[
            pltpu.VMEM((b_per_w,), jnp.int32),
            pltpu.VMEM((b_per_w,), jnp.int32),
            pltpu.VMEM((2, 4, SL, 128), jnp.float32),
            pltpu.VMEM((2, 4, SL, 128), jnp.float32),
            pltpu.VMEM((2, 4, SL, 128), jnp.float32),
            pltpu.VMEM((2, 4, SL, 128), jnp.float32),
            pltpu.VMEM((GW, 128), jnp.float32),
            pltpu.VMEM((GW, 128), jnp.float32),
            pltpu.VMEM((b_per_w,), jnp.float32),
            pltpu.SemaphoreType.DMA,
            pltpu.SemaphoreType.DMA,
        ],
    )
    def sc_kernel(stu_hbm, exer_hbm, th3_hbm, a3_hbm, b2_hbm, out_hbm,
                  sidx, eidx, thA, thB, aA, aB, bA, bB, res, semA, semB):
        wid = lax.axis_index("s") * NC + lax.axis_index("c")
        base = wid * b_per_w
        pltpu.sync_copy(stu_hbm.at[pl.ds(base, b_per_w)], sidx)
        pltpu.sync_copy(exer_hbm.at[pl.ds(base, b_per_w)], eidx)

        lanes = lax.broadcasted_iota(jnp.int32, (GW,), 0)

        def load_idx(ref, g):
            return ref[pl.ds(g * GW, GW)]

        def issue_group(g, th_buf, a_buf, b_buf, sem):
            vs = load_idx(sidx, g)
            ve = load_idx(eidx, g)
            pltpu.async_copy(b2_hbm.at[ve // 128], b_buf, sem)

            @plsc.parallel_loop(0, GW, unroll=2)
            def _(j):
                m = lanes == j
                sj = jnp.sum(jnp.where(m, vs, 0))
                ej = jnp.sum(jnp.where(m, ve, 0))
                s_start = (sj // GW) * GW
                e_start = (ej // GW) * GW
                half, slot = j // 8, (j % 8) * GW
                pltpu.async_copy(
                    th3_hbm.at[:, :, pl.ds(s_start, GW)],
                    th_buf.at[half, :, :, pl.ds(slot, GW)], sem)
                pltpu.async_copy(
                    a3_hbm.at[:, :, pl.ds(e_start, GW)],
                    a_buf.at[half, :, :, pl.ds(slot, GW)], sem)

        def drain_group(th_buf, a_buf, b_buf, sem):
            dummy3 = th3_hbm.at[:, :, pl.ds(0, 128)]
            for buf in (th_buf, a_buf):
                pltpu.make_async_copy(dummy3, buf.at[0], sem).wait()
                pltpu.make_async_copy(dummy3, buf.at[1], sem).wait()
            pltpu.make_async_copy(b2_hbm.at[pl.ds(0, GW)], b_buf, sem).wait()

        def compute_group(g, th_buf, a_buf, b_buf):
            vs = load_idx(sidx, g)
            ve = load_idx(eidx, g)
            half_v = lanes // 8
            s_lane = (lanes % 8) * GW + lax.rem(vs, GW)
            e_lane = (lanes % 8) * GW + lax.rem(ve, GW)
            b_off = lax.rem(ve, 128)
            zero_v = jnp.zeros((GW,), jnp.int32)

            def dbody(k, acc):
                acc0, acc1 = acc
                d = k * 2
                for dd, which in ((d, 0), (d + 1, 1)):
                    d0 = zero_v + dd // SL
                    d1 = zero_v + lax.rem(dd, SL)
                    th_v = plsc.load_gather(th_buf, [half_v, d0, d1, s_lane])
                    a_v = plsc.load_gather(a_buf, [half_v, d0, d1, e_lane])
                    asig = 1.0 / (1.0 + jnp.exp(-a_v))
                    if which == 0:
                        acc0 = acc0 + asig * th_v
                    else:
                        acc1 = acc1 + asig * th_v
                return acc0, acc1

            z = jnp.zeros((GW,), jnp.float32)
            acc0, acc1 = lax.fori_loop(0, D // 2, dbody, (z, z))
            acc = acc0 + acc1
            b_v = plsc.load_gather(b_buf, [lanes, b_off])
            logit = acc - b_v
            res[pl.ds(g * GW, GW)] = 1.0 / (1.0 + jnp.exp(-logit))

        issue_group(0, thA, aA, bA, semA)

        @pl.loop(0, (n_groups - 2) // 2)
        def _(i):
            g = i * 2
            issue_group(g + 1, thB, aB, bB, semB)
            drain_group(thA, aA, bA, semA)
            compute_group(g, thA, aA, bA)
            issue_group(g + 2, thA, aA, bA, semA)
            drain_group(thB, aB, bB, semB)
            compute_group(g + 1, thB, aB, bB)

        issue_group(n_groups - 1, thB, aB, bB, semB)
        drain_group(thA, aA, bA, semA)
        compute_group(n_groups - 2, thA, aA, bA)
        drain_group(thB, aB, bB, semB)
        compute_group(n_groups - 1, thB, aB, bB)

        pltpu.sync_copy(res, out_hbm.at[pl.ds(base, b_per_w)])

    return sc_kernel


def kernel(stu_id, input_exercise, theta_w, a_w, b_w):
    B = stu_id.shape[0]
    b_per_w = B // NW
    n_groups = b_per_w // GW
    stu1 = stu_id.astype(jnp.int32)
    exer1 = input_exercise.astype(jnp.int32)

    th3 = jnp.transpose(theta_w).reshape(4, SL, theta_w.shape[0])
    a3 = jnp.transpose(a_w).reshape(4, SL, a_w.shape[0])

    nb = b_w.shape[0]
    pad = (-nb) % 128
    b_flat = b_w.reshape(nb)
    if pad:
        b_flat = jnp.pad(b_flat, (0, pad))
    b2 = b_flat.reshape((nb + pad) // 128, 128)

    sc_kernel = _make_sc_kernel(B, b_per_w, n_groups)
    return sc_kernel(stu1, exer1, th3, a3, b2)


# speedup vs baseline: 1.2936x; 1.2936x over previous
import functools

import jax
import jax.numpy as jnp
from jax import lax
from jax.experimental import pallas as pl
from jax.experimental.pallas import tpu as pltpu
from jax.experimental.pallas import tpu_sc as plsc

NC = 2
NS = 16
NW = NC * NS
GW = 16
D = 32
SL = 8


def _make_sc_kernel(B, b_per_w, n_groups):
    mesh = plsc.VectorSubcoreMesh(core_axis_name="c", subcore_axis_name="s")

    @functools.partial(
        pl.kernel,
        mesh=mesh,
        compiler_params=pltpu.CompilerParams(
            use_tc_tiling_on_sc=True, needs_layout_passes=False),
        out_type=jax.ShapeDtypeStruct((B,), jnp.float32),
        scratch_types=[
            pltpu.VMEM((b_per_w,), jnp.int32),
            pltpu.VMEM((b_per_w,), jnp.int32),
            pltpu.VMEM((2, 4, SL, 128), jnp.float32),
            pltpu.VMEM((2, 4, SL, 128), jnp.float32),
            pltpu.VMEM((2, 4, SL, 128), jnp.float32),
            pltpu.VMEM((2, 4, SL, 128), jnp.float32),
            pltpu.VMEM((GW, 128), jnp.float32),
            pltpu.VMEM((GW, 128), jnp.float32),
            pltpu.VMEM((b_per_w,), jnp.float32),
            pltpu.SemaphoreType.DMA,
            pltpu.SemaphoreType.DMA,
        ],
    )
    def sc_kernel(stu_hbm, exer_hbm, th3_hbm, a3_hbm, b2_hbm, out_hbm,
                  sidx, eidx, thA, thB, aA, aB, bA, bB, res, semA, semB):
        wid = lax.axis_index("s") * NC + lax.axis_index("c")
        base = wid * b_per_w
        pltpu.sync_copy(stu_hbm.at[pl.ds(base, b_per_w)], sidx)
        pltpu.sync_copy(exer_hbm.at[pl.ds(base, b_per_w)], eidx)

        lanes = lax.broadcasted_iota(jnp.int32, (GW,), 0)

        def load_idx(ref, g):
            return ref[pl.ds(g * GW, GW)]

        def issue_group(g, th_buf, a_buf, b_buf, sem):
            vs = load_idx(sidx, g)
            ve = load_idx(eidx, g)
            pltpu.async_copy(b2_hbm.at[ve // 128], b_buf, sem)

            @pl.loop(0, GW)
            def _(j):
                m = lanes == j
                sj = jnp.sum(jnp.where(m, vs, 0))
                ej = jnp.sum(jnp.where(m, ve, 0))
                s_start = (sj // GW) * GW
                e_start = (ej // GW) * GW
                half, slot = j // 8, (j % 8) * GW
                pltpu.async_copy(
                    th3_hbm.at[:, :, pl.ds(s_start, GW)],
                    th_buf.at[half, :, :, pl.ds(slot, GW)], sem)
                pltpu.async_copy(
                    a3_hbm.at[:, :, pl.ds(e_start, GW)],
                    a_buf.at[half, :, :, pl.ds(slot, GW)], sem)

        def drain_group(th_buf, a_buf, b_buf, sem):
            dummy3 = th3_hbm.at[:, :, pl.ds(0, 128)]
            for buf in (th_buf, a_buf):
                pltpu.make_async_copy(dummy3, buf.at[0], sem).wait()
                pltpu.make_async_copy(dummy3, buf.at[1], sem).wait()
            pltpu.make_async_copy(b2_hbm.at[pl.ds(0, GW)], b_buf, sem).wait()

        def compute_group(g, th_buf, a_buf, b_buf):
            vs = load_idx(sidx, g)
            ve = load_idx(eidx, g)
            half_v = lanes // 8
            s_lane = (lanes % 8) * GW + lax.rem(vs, GW)
            e_lane = (lanes % 8) * GW + lax.rem(ve, GW)
            b_off = lax.rem(ve, 128)
            zero_v = jnp.zeros((GW,), jnp.int32)

            def dbody(k, acc):
                acc0, acc1 = acc
                d = k * 2
                for dd, which in ((d, 0), (d + 1, 1)):
                    d0 = zero_v + dd // SL
                    d1 = zero_v + lax.rem(dd, SL)
                    th_v = plsc.load_gather(th_buf, [half_v, d0, d1, s_lane])
                    a_v = plsc.load_gather(a_buf, [half_v, d0, d1, e_lane])
                    asig = 1.0 / (1.0 + jnp.exp(-a_v))
                    if which == 0:
                        acc0 = acc0 + asig * th_v
                    else:
                        acc1 = acc1 + asig * th_v
                return acc0, acc1

            z = jnp.zeros((GW,), jnp.float32)
            acc0, acc1 = lax.fori_loop(0, D // 2, dbody, (z, z))
            acc = acc0 + acc1
            b_v = plsc.load_gather(b_buf, [lanes, b_off])
            logit = acc - b_v
            res[pl.ds(g * GW, GW)] = 1.0 / (1.0 + jnp.exp(-logit))

        issue_group(0, thA, aA, bA, semA)

        @pl.loop(0, (n_groups - 2) // 2)
        def _(i):
            g = i * 2
            issue_group(g + 1, thB, aB, bB, semB)
            drain_group(thA, aA, bA, semA)
            compute_group(g, thA, aA, bA)
            issue_group(g + 2, thA, aA, bA, semA)
            drain_group(thB, aB, bB, semB)
            compute_group(g + 1, thB, aB, bB)

        issue_group(n_groups - 1, thB, aB, bB, semB)
        drain_group(thA, aA, bA, semA)
        compute_group(n_groups - 2, thA, aA, bA)
        drain_group(thB, aB, bB, semB)
        compute_group(n_groups - 1, thB, aB, bB)

        pltpu.sync_copy(res, out_hbm.at[pl.ds(base, b_per_w)])

    return sc_kernel


def kernel(stu_id, input_exercise, theta_w, a_w, b_w):
    B = stu_id.shape[0]
    b_per_w = B // NW
    n_groups = b_per_w // GW
    stu1 = stu_id.astype(jnp.int32)
    exer1 = input_exercise.astype(jnp.int32)

    th3 = jnp.transpose(theta_w).reshape(4, SL, theta_w.shape[0])
    a3 = jnp.transpose(a_w).reshape(4, SL, a_w.shape[0])

    nb = b_w.shape[0]
    pad = (-nb) % 128
    b_flat = b_w.reshape(nb)
    if pad:
        b_flat = jnp.pad(b_flat, (0, pad))
    b2 = b_flat.reshape((nb + pad) // 128, 128)

    sc_kernel = _make_sc_kernel(B, b_per_w, n_groups)
    return sc_kernel(stu1, exer1, th3, a3, b2)
